# baseline (speedup 1.0000x reference)
import functools

import jax
import jax.numpy as jnp
from jax import lax
from jax.experimental import pallas as pl
from jax.experimental.pallas import tpu as pltpu
from jax.experimental.pallas import tpu_sc as plsc

NW = 32
LANES = 16
NCHUNK = 6
GRP = 256


def _seg_sum_sc(table, src, dst, w, np_, n_edges):
    assert n_edges % GRP == 0
    ch = -(-n_edges // (NW * NCHUNK * GRP)) * GRP
    trips = ch // GRP
    mesh = plsc.VectorSubcoreMesh(core_axis_name="c", subcore_axis_name="s")

    @functools.partial(
        pl.kernel,
        out_type=jax.ShapeDtypeStruct((NW, np_), jnp.float32),
        mesh=mesh,
        compiler_params=pltpu.CompilerParams(needs_layout_passes=False),
        scratch_types=[
            pltpu.VMEM((np_,), jnp.float32),
            pltpu.VMEM((np_,), jnp.float32),
            pltpu.VMEM((2, ch), jnp.int32),
            pltpu.VMEM((2, ch), jnp.int32),
            pltpu.VMEM((2, ch), jnp.float32),
            pltpu.SemaphoreType.DMA,
            pltpu.SemaphoreType.DMA,
            pltpu.SemaphoreType.DMA,
        ],
    )
    def k(table_hbm, src_hbm, dst_hbm, w_hbm, out_hbm, tab_v, acc_v,
          src_v, dst_v, w_v, sem_t, sem_e0, sem_e1):
        wid = lax.axis_index("s") * 2 + lax.axis_index("c")
        sems = (sem_e0, sem_e1)
        td = pltpu.async_copy(table_hbm, tab_v, sem_t)

        def start(j, b):
            c = wid * NCHUNK + j
            raw = c * ch
            base = pl.multiple_of(jnp.clip(raw, 0, n_edges - ch), GRP)
            t0 = (raw - base) // GRP
            ds = pltpu.async_copy(src_hbm.at[pl.ds(base, ch)], src_v.at[b], sems[b])
            dd = pltpu.async_copy(dst_hbm.at[pl.ds(base, ch)], dst_v.at[b], sems[b])
            dw = pltpu.async_copy(w_hbm.at[pl.ds(base, ch)], w_v.at[b], sems[b])
            return (ds, dd, dw), t0

        pend, t0s = {}, {}
        pend[0], t0s[0] = start(0, 0)

        zeros = jnp.zeros((LANES,), jnp.float32)

        @plsc.parallel_loop(0, np_ // GRP)
        def _(i):
            for kk in range(GRP // LANES):
                acc_v[pl.ds(i * GRP + kk * LANES, LANES)] = zeros

        td.wait()

        for j in range(NCHUNK):
            b = j % 2
            for d in pend.pop(j):
                d.wait()
            if j + 1 < NCHUNK:
                pend[j + 1], t0s[j + 1] = start(j + 1, 1 - b)

            @plsc.parallel_loop(t0s[j], trips)
            def _(t, b=b):
                off = t * GRP
                for kk in range(GRP // LANES):
                    o = off + kk * LANES
                    sv = src_v[b, pl.ds(o, LANES)]
                    dv = dst_v[b, pl.ds(o, LANES)]
                    wv = w_v[b, pl.ds(o, LANES)]
                    xv = plsc.load_gather(tab_v, [sv])
                    plsc.addupdate_scatter(acc_v, [dv], xv * wv)

        pltpu.sync_copy(acc_v, out_hbm.at[wid])

    return k(table, src, dst, w)


def _split_tc(edge_index, n_edges):
    sb = 128 * 1024

    def body(e_ref, s_ref, d_ref):
        s_ref[...] = e_ref[0]
        d_ref[...] = e_ref[1]

    return pl.pallas_call(
        body,
        grid=(-(-n_edges // sb),),
        in_specs=[pl.BlockSpec((2, sb), lambda i: (0, i))],
        out_specs=[
            pl.BlockSpec((sb,), lambda i: (i,)),
            pl.BlockSpec((sb,), lambda i: (i,)),
        ],
        out_shape=[
            jax.ShapeDtypeStruct((n_edges,), jnp.int32),
            jax.ShapeDtypeStruct((n_edges,), jnp.int32),
        ],
    )(edge_index)


def _pq_tc(partials, x2d, wpre, w5, np_, nb):
    grid = np_ // nb
    dn = (((1,), (0,)), ((), ()))

    def body(part_ref, x_ref, wpre_ref, w5_ref, p_ref, q_ref):
        a = jnp.sum(part_ref[...], axis=0, keepdims=True)
        m = jnp.concatenate([a, x_ref[...], jnp.ones_like(a)], axis=0)
        h = jnp.maximum(
            lax.dot_general(wpre_ref[...], m, dn,
                            preferred_element_type=jnp.float32), 0.0)
        pq = lax.dot_general(w5_ref[...], h, dn,
                             preferred_element_type=jnp.float32)
        p_ref[...] = pq[0:1]
        q_ref[...] = pq[1:2]

    return pl.pallas_call(
        body,
        grid=(grid,),
        in_specs=[
            pl.BlockSpec((NW, nb), lambda i: (0, i)),
            pl.BlockSpec((1, nb), lambda i: (0, i)),
            pl.BlockSpec((64, 3), lambda i: (0, 0)),
            pl.BlockSpec((2, 64), lambda i: (0, 0)),
        ],
        out_specs=[
            pl.BlockSpec((1, nb), lambda i: (0, i)),
            pl.BlockSpec((1, nb), lambda i: (0, i)),
        ],
        out_shape=[
            jax.ShapeDtypeStruct((1, np_), jnp.float32),
            jax.ShapeDtypeStruct((1, np_), jnp.float32),
        ],
    )(partials, x2d, wpre, w5)


def _o_tc(partials, q, br5, np_, nb):
    grid = np_ // nb

    def body(part_ref, q_ref, b_ref, o_ref):
        s = jnp.sum(part_ref[...], axis=0, keepdims=True)
        o_ref[...] = s + q_ref[...] + b_ref[0, 0]

    return pl.pallas_call(
        body,
        grid=(grid,),
        in_specs=[
            pl.BlockSpec((NW, nb), lambda i: (0, i)),
            pl.BlockSpec((1, nb), lambda i: (0, i)),
            pl.BlockSpec((1, 1), lambda i: (0, 0)),
        ],
        out_specs=pl.BlockSpec((1, nb), lambda i: (0, i)),
        out_shape=jax.ShapeDtypeStruct((1, np_), jnp.float32),
    )(partials, q, br5)


def _mlp_tc(o8, W1, b1r, W2, b2r, W3, b3r, W4, b4r):
    KB = 10
    RB = 200
    dn = (((1,), (1,)), ((), ()))

    def body(o_ref, w1_ref, b1_ref, w2_ref, b2_ref, w3_ref, b3_ref,
             w4_ref, b4_ref, out_ref, z1s):
        i = pl.program_id(0)

        @pl.when(i < KB)
        def _():
            z = lax.dot_general(o_ref[...], w1_ref[...], dn,
                                preferred_element_type=jnp.float32)
            z1s[i] = jnp.maximum(z + b1_ref[0], 0.0)

        @pl.when(i == KB)
        def _():
            z1 = jnp.concatenate([z1s[j] for j in range(KB)], axis=1)
            h2 = jnp.maximum(
                lax.dot_general(z1, w2_ref[...], dn,
                                preferred_element_type=jnp.float32) + b2_ref[...], 0.0)
            h3 = jnp.maximum(
                lax.dot_general(h2, w3_ref[...], dn,
                                preferred_element_type=jnp.float32) + b3_ref[...], 0.0)
            lg = lax.dot_general(h3, w4_ref[...], dn,
                                 preferred_element_type=jnp.float32) + b4_ref[...]
            m = jnp.max(lg, axis=1, keepdims=True)
            sh = lg - m
            lse = jnp.log(jnp.sum(jnp.exp(sh), axis=1, keepdims=True))
            out_ref[...] = (sh - lse)[0:5, :]

    return pl.pallas_call(
        body,
        grid=(KB + 1,),
        in_specs=[
            pl.BlockSpec((8, 10000), lambda i: (0, 0)),
            pl.BlockSpec((RB, 10000), lambda i: (jnp.minimum(i, KB - 1), 0)),
            pl.BlockSpec((1, 1, RB), lambda i: (jnp.minimum(i, KB - 1), 0, 0)),
            pl.BlockSpec((400, 2000), lambda i: (0, 0)),
            pl.BlockSpec((1, 400), lambda i: (0, 0)),
            pl.BlockSpec((80, 400), lambda i: (0, 0)),
            pl.BlockSpec((1, 80), lambda i: (0, 0)),
            pl.BlockSpec((10, 80), lambda i: (0, 0)),
            pl.BlockSpec((1, 10), lambda i: (0, 0)),
        ],
        out_specs=pl.BlockSpec((5, 10), lambda i: (0, 0)),
        out_shape=jax.ShapeDtypeStruct((5, 10), jnp.float32),
        scratch_shapes=[pltpu.VMEM((KB, 8, RB), jnp.float32)],
    )(o8, W1, b1r, W2, b2r, W3, b3r, W4, b4r)


def kernel(x, edge_index, edge_weight, batch, W_rel1, b_rel1, W_root1,
           W_rel5, b_rel5, W_root5, W1, b1, W2, b2, W3, b3, W4, b4):
    N = x.shape[0]
    E = edge_index.shape[1]
    NP = -(-N // 1024) * 1024

    src, dst = _split_tc(edge_index, E)
    w = edge_weight

    xf = jnp.concatenate([x[:, 0], jnp.zeros((NP - N,), jnp.float32)])
    nb = NP // 4

    parts1 = _seg_sum_sc(xf, src, dst, w, NP, E)
    wpre = jnp.concatenate([W_rel1, W_root1, b_rel1.reshape(64, 1)], axis=1)
    w5 = jnp.concatenate([W_rel5, W_root5], axis=0)
    p, q = _pq_tc(parts1, xf.reshape(1, NP), wpre, w5, NP, nb)
    parts2 = _seg_sum_sc(p.reshape(NP), src, dst, w, NP, E)
    o = _o_tc(parts2, q, b_rel5.reshape(1, 1), NP, nb)

    o5 = o[0, :N].reshape(5, N // 5)
    o8 = jnp.concatenate([o5, jnp.zeros((3, N // 5), jnp.float32)], axis=0)
    out = _mlp_tc(o8, W1, b1.reshape(10, 1, 200), W2, b2.reshape(1, 400),
                  W3, b3.reshape(1, 80), W4, b4.reshape(1, 10))
    return out


# speedup vs baseline: 1.0849x; 1.0849x over previous
import functools

import jax
import jax.numpy as jnp
from jax import lax
from jax.experimental import pallas as pl
from jax.experimental.pallas import tpu as pltpu
from jax.experimental.pallas import tpu_sc as plsc

NW = 32
LANES = 16
NCHUNK = 6
GRP = 128


def _seg_sum_sc(table, src, dst, w, np_, n_edges):
    assert n_edges % GRP == 0
    ch = -(-n_edges // (NW * NCHUNK * GRP)) * GRP
    trips = ch // GRP
    mesh = plsc.VectorSubcoreMesh(core_axis_name="c", subcore_axis_name="s")

    @functools.partial(
        pl.kernel,
        out_type=jax.ShapeDtypeStruct((NW, np_), jnp.float32),
        mesh=mesh,
        compiler_params=pltpu.CompilerParams(needs_layout_passes=False),
        scratch_types=[
            pltpu.VMEM((np_,), jnp.float32),
            pltpu.VMEM((np_,), jnp.float32),
            pltpu.VMEM((2, ch), jnp.int32),
            pltpu.VMEM((2, ch), jnp.int32),
            pltpu.VMEM((2, ch), jnp.float32),
            pltpu.SemaphoreType.DMA,
            pltpu.SemaphoreType.DMA,
            pltpu.SemaphoreType.DMA,
        ],
    )
    def k(table_hbm, src_hbm, dst_hbm, w_hbm, out_hbm, tab_v, acc_v,
          src_v, dst_v, w_v, sem_t, sem_e0, sem_e1):
        wid = lax.axis_index("s") * 2 + lax.axis_index("c")
        sems = (sem_e0, sem_e1)
        td = pltpu.async_copy(table_hbm, tab_v, sem_t)

        def start(j, b):
            c = wid * NCHUNK + j
            raw = c * ch
            base = pl.multiple_of(jnp.clip(raw, 0, n_edges - ch), GRP)
            t0 = (raw - base) // GRP
            ds = pltpu.async_copy(src_hbm.at[pl.ds(base, ch)], src_v.at[b], sems[b])
            dd = pltpu.async_copy(dst_hbm.at[pl.ds(base, ch)], dst_v.at[b], sems[b])
            dw = pltpu.async_copy(w_hbm.at[pl.ds(base, ch)], w_v.at[b], sems[b])
            return (ds, dd, dw), t0

        pend, t0s = {}, {}
        pend[0], t0s[0] = start(0, 0)

        zeros = jnp.zeros((LANES,), jnp.float32)

        @plsc.parallel_loop(0, np_ // GRP)
        def _(i):
            for kk in range(GRP // LANES):
                acc_v[pl.ds(i * GRP + kk * LANES, LANES)] = zeros

        td.wait()

        for j in range(NCHUNK):
            b = j % 2
            for d in pend.pop(j):
                d.wait()
            if j + 1 < NCHUNK:
                pend[j + 1], t0s[j + 1] = start(j + 1, 1 - b)

            @plsc.parallel_loop(t0s[j], trips)
            def _(t, b=b):
                off = t * GRP
                for kk in range(GRP // LANES):
                    o = off + kk * LANES
                    sv = src_v[b, pl.ds(o, LANES)]
                    dv = dst_v[b, pl.ds(o, LANES)]
                    wv = w_v[b, pl.ds(o, LANES)]
                    xv = plsc.load_gather(tab_v, [sv])
                    plsc.addupdate_scatter(acc_v, [dv], xv * wv)

        pltpu.sync_copy(acc_v, out_hbm.at[wid])

    return k(table, src, dst, w)


def _split_tc(edge_index, n_edges):
    sb = 128 * 1024

    def body(e_ref, s_ref, d_ref):
        s_ref[...] = e_ref[0]
        d_ref[...] = e_ref[1]

    return pl.pallas_call(
        body,
        grid=(-(-n_edges // sb),),
        in_specs=[pl.BlockSpec((2, sb), lambda i: (0, i))],
        out_specs=[
            pl.BlockSpec((sb,), lambda i: (i,)),
            pl.BlockSpec((sb,), lambda i: (i,)),
        ],
        out_shape=[
            jax.ShapeDtypeStruct((n_edges,), jnp.int32),
            jax.ShapeDtypeStruct((n_edges,), jnp.int32),
        ],
    )(edge_index)


def _pq_tc(partials, x2d, wpre, w5, np_, nb):
    grid = np_ // nb
    dn = (((1,), (0,)), ((), ()))

    def body(part_ref, x_ref, wpre_ref, w5_ref, p_ref, q_ref):
        a = jnp.sum(part_ref[...], axis=0, keepdims=True)
        m = jnp.concatenate([a, x_ref[...], jnp.ones_like(a)], axis=0)
        h = jnp.maximum(
            lax.dot_general(wpre_ref[...], m, dn,
                            preferred_element_type=jnp.float32), 0.0)
        pq = lax.dot_general(w5_ref[...], h, dn,
                             preferred_element_type=jnp.float32)
        p_ref[...] = pq[0:1]
        q_ref[...] = pq[1:2]

    return pl.pallas_call(
        body,
        grid=(grid,),
        in_specs=[
            pl.BlockSpec((NW, nb), lambda i: (0, i)),
            pl.BlockSpec((1, nb), lambda i: (0, i)),
            pl.BlockSpec((64, 3), lambda i: (0, 0)),
            pl.BlockSpec((2, 64), lambda i: (0, 0)),
        ],
        out_specs=[
            pl.BlockSpec((1, nb), lambda i: (0, i)),
            pl.BlockSpec((1, nb), lambda i: (0, i)),
        ],
        out_shape=[
            jax.ShapeDtypeStruct((1, np_), jnp.float32),
            jax.ShapeDtypeStruct((1, np_), jnp.float32),
        ],
    )(partials, x2d, wpre, w5)


def _mlp_tc(partials, q, br5, n, W1, b1r, W2, b2r, W3, b3r, W4, b4r):
    KB = 10
    RB = 200
    nr = n // 5
    dn = (((1,), (1,)), ((), ()))

    def body(part_ref, q_ref, b5_ref, w1_ref, b1_ref, w2_ref, b2_ref,
             w3_ref, b3_ref, w4_ref, b4_ref, out_ref, z1s, o8s):
        i = pl.program_id(0)

        @pl.when(i == 0)
        def _():
            o = jnp.sum(part_ref[...], axis=0, keepdims=True) + q_ref[...] + b5_ref[0, 0]
            for r in range(5):
                o8s[r:r + 1, :] = o[:, r * nr:(r + 1) * nr]
            o8s[5:8] = jnp.zeros((3, nr), jnp.float32)

        @pl.when(i < KB)
        def _():
            z = lax.dot_general(o8s[...], w1_ref[...], dn,
                                preferred_element_type=jnp.float32)
            z1s[i] = jnp.maximum(z + b1_ref[0], 0.0)

        @pl.when(i == KB)
        def _():
            z1 = jnp.concatenate([z1s[j] for j in range(KB)], axis=1)
            h2 = jnp.maximum(
                lax.dot_general(z1, w2_ref[...], dn,
                                preferred_element_type=jnp.float32) + b2_ref[...], 0.0)
            h3 = jnp.maximum(
                lax.dot_general(h2, w3_ref[...], dn,
                                preferred_element_type=jnp.float32) + b3_ref[...], 0.0)
            lg = lax.dot_general(h3, w4_ref[...], dn,
                                 preferred_element_type=jnp.float32) + b4_ref[...]
            m = jnp.max(lg, axis=1, keepdims=True)
            sh = lg - m
            lse = jnp.log(jnp.sum(jnp.exp(sh), axis=1, keepdims=True))
            out_ref[...] = (sh - lse)[0:5, :]

    np_ = partials.shape[1]
    return pl.pallas_call(
        body,
        grid=(KB + 1,),
        in_specs=[
            pl.BlockSpec((NW, np_), lambda i: (0, 0)),
            pl.BlockSpec((1, np_), lambda i: (0, 0)),
            pl.BlockSpec((1, 1), lambda i: (0, 0)),
            pl.BlockSpec((RB, 10000), lambda i: (jnp.minimum(i, KB - 1), 0)),
            pl.BlockSpec((1, 1, RB), lambda i: (jnp.minimum(i, KB - 1), 0, 0)),
            pl.BlockSpec((400, 2000), lambda i: (0, 0)),
            pl.BlockSpec((1, 400), lambda i: (0, 0)),
            pl.BlockSpec((80, 400), lambda i: (0, 0)),
            pl.BlockSpec((1, 80), lambda i: (0, 0)),
            pl.BlockSpec((10, 80), lambda i: (0, 0)),
            pl.BlockSpec((1, 10), lambda i: (0, 0)),
        ],
        out_specs=pl.BlockSpec((5, 10), lambda i: (0, 0)),
        out_shape=jax.ShapeDtypeStruct((5, 10), jnp.float32),
        scratch_shapes=[pltpu.VMEM((KB, 8, RB), jnp.float32),
                        pltpu.VMEM((8, nr), jnp.float32)],
    )(partials, q, br5, W1, b1r, W2, b2r, W3, b3r, W4, b4r)


def kernel(x, edge_index, edge_weight, batch, W_rel1, b_rel1, W_root1,
           W_rel5, b_rel5, W_root5, W1, b1, W2, b2, W3, b3, W4, b4):
    N = x.shape[0]
    E = edge_index.shape[1]
    NP = -(-N // 1024) * 1024

    src, dst = _split_tc(edge_index, E)
    w = edge_weight

    xf = jnp.concatenate([x[:, 0], jnp.zeros((NP - N,), jnp.float32)])
    nb = NP // 4

    parts1 = _seg_sum_sc(xf, src, dst, w, NP, E)
    wpre = jnp.concatenate([W_rel1, W_root1, b_rel1.reshape(64, 1)], axis=1)
    w5 = jnp.concatenate([W_rel5, W_root5], axis=0)
    p, q = _pq_tc(parts1, xf.reshape(1, NP), wpre, w5, NP, nb)
    parts2 = _seg_sum_sc(p.reshape(NP), src, dst, w, NP, E)
    out = _mlp_tc(parts2, q, b_rel5.reshape(1, 1), N,
                  W1, b1.reshape(10, 1, 200), W2, b2.reshape(1, 400),
                  W3, b3.reshape(1, 80), W4, b4.reshape(1, 10))
    return out
